# Initial kernel scaffold; baseline (speedup 1.0000x reference)
import jax
import jax.numpy as jnp
from jax.experimental import pallas as pl


def kernel(inputs, params):
    raise NotImplementedError("write your pallas kernel here")


# speedup vs baseline: 1.1505x; 1.1505x over previous
import functools
import math

import jax
import jax.numpy as jnp
from jax.experimental import pallas as pl
from jax.experimental.pallas import tpu as pltpu
from jax.experimental.pallas import tpu_sc as plsc

B, S, D = 8, 512, 64
K_CODE = 8192
HEADS = 8
DH = D // HEADS
KERNELS = (3, 5, 7, 9)
MAXPAD = max(KERNELS) // 2
TOK = B * S
CB_CHUNK = 2048


def _encoder_body(x_ref, pos_ref, wt_ref, cb4_ref, pjw_ref, pjb_ref,
                  aiw_ref, aib_ref, aow_ref, aob_ref, out_ref):
    x = x_ref[0] + pos_ref[...]
    zeros = jnp.zeros((MAXPAD, D), jnp.float32)
    xpad = jnp.concatenate([zeros, x, zeros], axis=0)

    ys = []
    off = 0
    for i, k in enumerate(KERNELS):
        p = k // 2
        acc = None
        for t in range(k):
            shift = xpad[MAXPAD - p + t:MAXPAD - p + t + S, :]
            part = jax.lax.dot(shift, wt_ref[off + t])
            acc = part if acc is None else acc + part
        off += k
        y = acc + cb4_ref[i][None, :]
        ys.append(jnp.maximum(y, 0.0))
    multi = jnp.concatenate(ys, axis=1)

    projected = jax.lax.dot(multi, pjw_ref[...]) + pjb_ref[...]
    qkv = jax.lax.dot(projected, aiw_ref[...]) + aib_ref[...]
    q = qkv[:, :D]
    k_ = qkv[:, D:2 * D]
    v = qkv[:, 2 * D:]

    scale = jnp.sqrt(jnp.float32(DH))
    os_ = []
    for h in range(HEADS):
        sl = slice(h * DH, (h + 1) * DH)
        qh, kh, vh = q[:, sl], k_[:, sl], v[:, sl]
        logits = jax.lax.dot_general(
            qh, kh, (((1,), (1,)), ((), ()))) / scale
        mx = jnp.max(logits, axis=-1, keepdims=True)
        e = jnp.exp(logits - mx)
        s = jnp.sum(e, axis=-1, keepdims=True)
        os_.append(jax.lax.dot(e, vh) / s)
    o = jnp.concatenate(os_, axis=1)

    attended = jax.lax.dot(o, aow_ref[...]) + aob_ref[...]
    out_ref[0] = projected + attended


def _run_encoder(x, params):
    wt = jnp.concatenate(
        [jnp.transpose(w, (2, 1, 0)) for w in params['conv_w']], axis=0)
    cb4 = jnp.stack(params['conv_b'], axis=0)
    pjw = params['proj_w'].T
    aiw = params['attn_in_w'].T
    aow = params['attn_out_w'].T

    specs = [
        pl.BlockSpec((1, S, D), lambda b: (b, 0, 0)),
        pl.BlockSpec((S, D), lambda b: (0, 0)),
        pl.BlockSpec((sum(KERNELS), D, D), lambda b: (0, 0, 0)),
        pl.BlockSpec((len(KERNELS), D), lambda b: (0, 0)),
        pl.BlockSpec((4 * D, D), lambda b: (0, 0)),
        pl.BlockSpec((1, D), lambda b: (0, 0)),
        pl.BlockSpec((D, 3 * D), lambda b: (0, 0)),
        pl.BlockSpec((1, 3 * D), lambda b: (0, 0)),
        pl.BlockSpec((D, D), lambda b: (0, 0)),
        pl.BlockSpec((1, D), lambda b: (0, 0)),
    ]
    return pl.pallas_call(
        _encoder_body,
        grid=(B,),
        in_specs=specs,
        out_specs=pl.BlockSpec((1, S, D), lambda b: (b, 0, 0)),
        out_shape=jax.ShapeDtypeStruct((B, S, D), jnp.float32),
    )(x, params['pos'], wt, cb4, pjw, params['proj_b'][None, :],
      aiw, params['attn_in_b'][None, :], aow, params['attn_out_b'][None, :])


def _vq_body(flat_ref, x2_ref, cb_ref, idx_ref):
    flat = flat_ref[0]
    x2 = x2_ref[0]
    best_d = jnp.full((S,), jnp.inf, jnp.float32)
    best_i = jnp.zeros((S,), jnp.int32)
    for c0 in range(0, K_CODE, CB_CHUNK):
        cb = cb_ref[c0:c0 + CB_CHUNK, :]
        c2 = jnp.sum(cb ** 2, axis=1)
        m = jax.lax.dot_general(flat, cb, (((1,), (1,)), ((), ())))
        d = (x2 + c2[None, :]) - 2.0 * m
        loc_d = jnp.min(d, axis=1)
        iota = jax.lax.broadcasted_iota(jnp.int32, d.shape, 1)
        cand = jnp.where(d == loc_d[:, None], iota, K_CODE)
        loc_i = jnp.min(cand, axis=1) + c0
        upd = loc_d < best_d
        best_d = jnp.where(upd, loc_d, best_d)
        best_i = jnp.where(upd, loc_i, best_i)
    idx_ref[0, 0] = best_i


def _run_vq(flat, x2, codebook):
    return pl.pallas_call(
        _vq_body,
        grid=(TOK // S,),
        in_specs=[
            pl.BlockSpec((1, S, D), lambda t: (t, 0, 0)),
            pl.BlockSpec((1, S, 1), lambda t: (t, 0, 0)),
            pl.BlockSpec((K_CODE, D), lambda t: (0, 0)),
        ],
        out_specs=pl.BlockSpec((1, 1, S), lambda t: (t, 0, 0)),
        out_shape=jax.ShapeDtypeStruct((TOK // S, 1, S), jnp.int32),
    )(flat.reshape(TOK // S, S, D), x2.reshape(TOK // S, S, 1), codebook)


_GATHER_W = 128


def _sc_gather(codebook, idx):
    cb_wide = jnp.pad(codebook, ((0, 0), (0, 128 - D)))
    indices = idx.reshape(1, TOK)
    mesh = plsc.VectorSubcoreMesh(
        core_axis_name="core", subcore_axis_name="subcore")

    @functools.partial(
        pl.kernel,
        out_type=jax.ShapeDtypeStruct((TOK, 128), codebook.dtype),
        mesh=mesh)
    def kern(x_hbm, i_hbm, o_hbm):
        def body(i_vmem, o_vmem):
            pltpu.sync_copy(x_hbm.at[i_vmem.at[0]], o_vmem)

        pltpu.emit_pipeline(
            body,
            grid=(TOK // _GATHER_W,),
            in_specs=[pl.BlockSpec((1, _GATHER_W), index_map=lambda i: (0, i))],
            out_specs=[pl.BlockSpec((_GATHER_W, 128), index_map=lambda i: (i, 0))],
            core_axis_name='subcore',
            dimension_semantics=(pltpu.PARALLEL,),
        )(i_hbm, o_hbm)

    return kern(cb_wide, indices)[:, :D]


def _finish_body(e_ref, q_ref, hw_ref, qst_ref, bits_ref, loss_ref):
    e = e_ref[...]
    q = q_ref[...]
    diff = q - e
    qst = e + diff
    qst_ref[...] = qst
    loss_ref[...] = (1.25 * jnp.mean(diff * diff)).reshape(1, 1)
    h = jax.lax.dot_general(qst, hw_ref[...], (((1,), (1,)), ((), ())))
    bits_ref[...] = (h > 0.0).astype(jnp.float32)


def _run_finish(flat, quant, hash_ws):
    hw = jnp.concatenate(hash_ws, axis=0)
    return pl.pallas_call(
        _finish_body,
        in_specs=[
            pl.BlockSpec((TOK, D), lambda: (0, 0)),
            pl.BlockSpec((TOK, D), lambda: (0, 0)),
            pl.BlockSpec((D, D), lambda: (0, 0)),
        ],
        out_specs=[
            pl.BlockSpec((TOK, D), lambda: (0, 0)),
            pl.BlockSpec((TOK, D), lambda: (0, 0)),
            pl.BlockSpec((1, 1), lambda: (0, 0)),
        ],
        out_shape=[
            jax.ShapeDtypeStruct((TOK, D), jnp.float32),
            jax.ShapeDtypeStruct((TOK, D), jnp.float32),
            jax.ShapeDtypeStruct((1, 1), jnp.float32),
        ],
    )(flat, quant, hw)


def kernel(inputs, params):
    encoded = _run_encoder(inputs, params)
    flat = encoded.reshape(TOK, D)
    x2 = jnp.sum(flat ** 2, axis=1, keepdims=True)
    idx = _run_vq(flat, x2, params['codebook'])
    quant = _sc_gather(params['codebook'], idx.reshape(TOK))
    qst, bits, loss = _run_finish(flat, quant, params['hash_w'])
    hash_codes = bits.reshape(B, S, len(params['hash_w']), -1)
    return (qst.reshape(B, S, D), loss.reshape(()), idx.reshape(B, S),
            hash_codes)
